# Initial kernel scaffold; baseline (speedup 1.0000x reference)
import jax
import jax.numpy as jnp
from jax.experimental import pallas as pl


def kernel(initial_llrs, edge_index, cycle_mask, att_W, att_b, min_sum_scaler, cycle_penalty):
    raise NotImplementedError("write your pallas kernel here")


# speedup vs baseline: 133.8317x; 133.8317x over previous
import jax
import jax.numpy as jnp
from jax import lax
from jax.experimental import pallas as pl
from jax.experimental.pallas import tpu as pltpu
from jax.experimental.pallas import tpu_sc as plsc

_B = 4096
_NVARS = 128
_NUM_NODES = 256
_NN = _B * _NUM_NODES
_E = 8388608
_NITER = 10
_NC = 2
_NS = 16
_L = 16
_HALF = _NN // 2
_CHUNK = 2048
_EPT = _E // _NS
_NCHUNKS = _EPT // _CHUNK
_XW = _NN // _NS
_AW = _HALF // _NS


def _iter_body(x_hbm, base_hbm, src_hbm, dst_hbm, cm_hbm, par_hbm,
               xout_hbm,
               x_sp, acc_sp, srcv, dstv, cmv, xsv, xdv, msgv, dlv, parv):
  c = lax.axis_index("c")
  s = lax.axis_index("s")

  pltpu.sync_copy(x_hbm.at[pl.ds(s * _XW, _XW)], x_sp.at[pl.ds(s * _XW, _XW)])
  pltpu.sync_copy(base_hbm.at[pl.ds(c * _HALF + s * _AW, _AW)],
                  acc_sp.at[pl.ds(s * _AW, _AW)])
  pltpu.sync_copy(par_hbm, parv)
  plsc.subcore_barrier()

  w0 = parv[pl.ds(0, _L)]
  w1 = parv[pl.ds(16, _L)]
  w2 = parv[pl.ds(32, _L)]
  bb = parv[pl.ds(48, _L)]
  pen = parv[pl.ds(64, _L)]
  scal = parv[pl.ds(80, _L)]

  @pl.loop(0, _NCHUNKS)
  def _chunks(k):
    e0 = s * _EPT + k * _CHUNK
    pltpu.sync_copy(src_hbm.at[pl.ds(e0, _CHUNK)], srcv)
    pltpu.sync_copy(dst_hbm.at[pl.ds(e0, _CHUNK)], dstv)
    pltpu.sync_copy(cm_hbm.at[pl.ds(e0, _CHUNK)], cmv)
    pltpu.sync_copy(x_sp.at[srcv], xsv)
    pltpu.sync_copy(x_sp.at[dstv], xdv)

    @pl.loop(0, _CHUNK // _L)
    def _vec(i):
      sl = pl.ds(i * _L, _L)
      xs = xsv[sl]
      xd = xdv[sl]
      cmx = cmv[sl]
      dd = dstv[sl]
      r = xs * w0 + xd * w1 + cmx * w2 + bb
      r = jnp.maximum(r, r * jnp.float32(0.01))
      r = r + cmx * pen
      a = jnp.float32(1.0) / (jnp.float32(1.0) + jnp.exp(-r))
      m = xs * a * scal
      ok = lax.shift_right_logical(dd, 19) == c
      msgv[sl] = jnp.where(ok, m, jnp.float32(0.0))
      dlv[sl] = lax.bitwise_and(dd, _HALF - 1)

    pltpu.sync_copy(msgv, acc_sp.at[dlv], add=True)

  plsc.subcore_barrier()
  pltpu.sync_copy(acc_sp.at[pl.ds(s * _AW, _AW)],
                  xout_hbm.at[pl.ds(c * _HALF + s * _AW, _AW)])


_decode_iter = pl.kernel(
    _iter_body,
    out_type=jax.ShapeDtypeStruct((_NN,), jnp.float32),
    mesh=plsc.VectorSubcoreMesh(core_axis_name="c", subcore_axis_name="s",
                                num_cores=_NC, num_subcores=_NS),
    scratch_types=[
        pltpu.VMEM_SHARED((_NN,), jnp.float32),
        pltpu.VMEM_SHARED((_HALF,), jnp.float32),
        pltpu.VMEM((_CHUNK,), jnp.int32),
        pltpu.VMEM((_CHUNK,), jnp.int32),
        pltpu.VMEM((_CHUNK,), jnp.float32),
        pltpu.VMEM((_CHUNK,), jnp.float32),
        pltpu.VMEM((_CHUNK,), jnp.float32),
        pltpu.VMEM((_CHUNK,), jnp.float32),
        pltpu.VMEM((_CHUNK,), jnp.int32),
        pltpu.VMEM((6 * _L,), jnp.float32),
    ],
)


def kernel(initial_llrs, edge_index, cycle_mask, att_W, att_b,
           min_sum_scaler, cycle_penalty):
  base = jnp.concatenate(
      [initial_llrs,
       jnp.zeros((_B, _NUM_NODES - _NVARS), initial_llrs.dtype)],
      axis=1).reshape(-1)
  src = edge_index[0]
  dst = edge_index[1]
  p = jnp.stack([att_W[:, 0, 0], att_W[:, 0, 1], att_W[:, 0, 2],
                 att_b[:, 0], cycle_penalty[:, 0], min_sum_scaler[:, 0]],
                axis=1)
  params = jnp.broadcast_to(p[:, :, None],
                            (_NITER, 6, _L)).reshape(_NITER, 6 * _L)
  params = params.astype(jnp.float32)
  x = base
  outs = []
  for i in range(_NITER):
    x = _decode_iter(x, base, src, dst, cycle_mask, params[i])
    outs.append(x.reshape(_B, _NUM_NODES)[:, :_NVARS])
  return tuple(outs)
